# baseline (speedup 1.0000x reference)
import functools

import jax
import jax.numpy as jnp
import numpy as np
from jax import lax
from jax.experimental import pallas as pl
from jax.experimental.pallas import tpu as pltpu
from jax.experimental.pallas import tpu_sc as plsc


def _disabled_tofs(tof_count, min_c, max_c, neighbor_p, seed=0):
    rng = np.random.default_rng(seed)
    count = int(rng.integers(min_c, max_c + 1))
    tof_list = rng.permutation(tof_count)
    first = int(rng.integers(1, tof_count))
    disabled = [first]
    tof_list = tof_list[tof_list != first]
    for _ in range(count - 1):
        r = float(rng.random())
        if r < neighbor_p:
            if r < neighbor_p / 2.0:
                offsets = (1, -1)
            else:
                offsets = (tof_count // 2, -(tof_count // 2))
            appended = False
            for d in list(disabled):
                for off in offsets:
                    cand = d + off
                    if cand in tof_list:
                        tof_list = tof_list[tof_list != cand]
                        disabled.append(int(cand))
                        appended = True
                        break
                if appended:
                    break
            if not appended:
                new = int(tof_list[0])
                tof_list = tof_list[tof_list != new]
                disabled.append(new)
        else:
            new = int(tof_list[0])
            tof_list = tof_list[tof_list != new]
            disabled.append(new)
    return sorted(int(x) for x in disabled)


_ROWS, _COLS = 16384, 2048
_NW = 32
_CH = 16
_NBUF = 2
_RA = 1
_N = _ROWS // (_NW * _CH)


@functools.cache
def _build(tof_count):
    disabled = _disabled_tofs(tof_count, 1, 3, 0.5)
    mesh = plsc.VectorSubcoreMesh(core_axis_name="c", subcore_axis_name="s")

    @functools.partial(
        pl.kernel,
        mesh=mesh,
        out_type=jax.ShapeDtypeStruct((_ROWS, _COLS), jnp.float32),
        scratch_types=(
            [pltpu.VMEM((_CH, _COLS), jnp.float32) for _ in range(_NBUF)]
            + [pltpu.SemaphoreType.DMA for _ in range(2 * _NBUF)]
        ),
    )
    def k(img_hbm, out_hbm, *rest):
        bufs = rest[:_NBUF]
        isems = rest[_NBUF:2 * _NBUF]
        osems = rest[2 * _NBUF:3 * _NBUF]
        wid = lax.axis_index("s") * 2 + lax.axis_index("c")
        iota = lax.iota(jnp.int32, 16)

        def in_cp(i, b):
            r = pl.ds((i * _NW + wid) * _CH, _CH)
            return pltpu.make_async_copy(img_hbm.at[r, :], bufs[b], isems[b])

        def out_cp(i, b):
            r = pl.ds((i * _NW + wid) * _CH, _CH)
            return pltpu.make_async_copy(bufs[b], out_hbm.at[r, :], osems[b])

        for p in range(_RA):
            in_cp(p, p).start()

        def body(g, carry):
            for b in range(_NBUF):
                i = g * _NBUF + b
                j = i + _RA
                bj = (b + _RA) % _NBUF

                @pl.when(j < _N)
                def _():
                    @pl.when(j >= _NBUF)
                    def _():
                        out_cp(j - _NBUF, bj).wait()
                    in_cp(j, bj).start()

                in_cp(i, b).wait()
                for r in range(_CH):
                    for c in disabled:
                        w = (c // 16) * 16
                        lane = c % 16
                        v = bufs[b][r, pl.ds(w, 16)]
                        bufs[b][r, pl.ds(w, 16)] = jnp.where(
                            iota == lane, 0.0, v)
                out_cp(i, b).start()
            return carry

        lax.fori_loop(0, _N // _NBUF, body, 0)
        for b in range(_NBUF):
            out_cp(_N - _NBUF + b, b).wait()

    return k


def kernel(img):
    return _build(img.shape[-1])(img)


# speedup vs baseline: 1.0429x; 1.0429x over previous
import functools

import jax
import jax.numpy as jnp
import numpy as np
from jax import lax
from jax.experimental import pallas as pl
from jax.experimental.pallas import tpu as pltpu
from jax.experimental.pallas import tpu_sc as plsc


def _disabled_tofs(tof_count, min_c, max_c, neighbor_p, seed=0):
    rng = np.random.default_rng(seed)
    count = int(rng.integers(min_c, max_c + 1))
    tof_list = rng.permutation(tof_count)
    first = int(rng.integers(1, tof_count))
    disabled = [first]
    tof_list = tof_list[tof_list != first]
    for _ in range(count - 1):
        r = float(rng.random())
        if r < neighbor_p:
            if r < neighbor_p / 2.0:
                offsets = (1, -1)
            else:
                offsets = (tof_count // 2, -(tof_count // 2))
            appended = False
            for d in list(disabled):
                for off in offsets:
                    cand = d + off
                    if cand in tof_list:
                        tof_list = tof_list[tof_list != cand]
                        disabled.append(int(cand))
                        appended = True
                        break
                if appended:
                    break
            if not appended:
                new = int(tof_list[0])
                tof_list = tof_list[tof_list != new]
                disabled.append(new)
        else:
            new = int(tof_list[0])
            tof_list = tof_list[tof_list != new]
            disabled.append(new)
    return sorted(int(x) for x in disabled)


_ROWS, _COLS = 16384, 2048
_NW = 32
_NS = 16
_CH = 8
_NG = _ROWS // (_NW * _CH)
_NA = _NG // 2
_NB = _NG - _NA


@functools.cache
def _build(tof_count):
    disabled = _disabled_tofs(tof_count, 1, 3, 0.5)
    windows = sorted({(c // 128) * 128 for c in disabled})
    groups = {w: sorted({((c - w) // 16) * 16 for c in disabled
                         if (c // 128) * 128 == w}) for w in windows}
    lanes = {w: {g: [c - w - g for c in disabled
                     if (c // 128) * 128 == w and ((c - w) // 16) * 16 == g]
                 for g in groups[w]} for w in windows}
    nwin = len(windows)
    mesh = plsc.VectorSubcoreMesh(core_axis_name="c", subcore_axis_name="s")

    @functools.partial(
        pl.kernel,
        mesh=mesh,
        out_type=jax.ShapeDtypeStruct((_ROWS, _COLS), jnp.float32),
        scratch_types=(
            [pltpu.VMEM_SHARED((_NS, 2, _CH, _COLS), jnp.float32)]
            + [pltpu.VMEM((_CH, _COLS), jnp.float32) for _ in range(2)]
            + [pltpu.VMEM((_CH, 128), jnp.float32) for _ in range(nwin)]
            + [pltpu.SemaphoreType.DMA for _ in range(8)]
        ),
    )
    def k(img_hbm, out_hbm, spm, *rest):
        tbufs = rest[:2]
        fbufs = rest[2:2 + nwin]
        sems = rest[2 + nwin:]
        ia_sems, oa_sems = sems[0:2], sems[2:4]
        ib_sems, ob_sems = sems[4:6], sems[6:8]
        sid = lax.axis_index("s")
        wid = sid * 2 + lax.axis_index("c")
        iota = lax.iota(jnp.int32, 16)

        def rows_a(i):
            return pl.ds(((2 * i) * _NW + wid) * _CH, _CH)

        def rows_b(i):
            return pl.ds(((2 * i + 1) * _NW + wid) * _CH, _CH)

        def in_a(i, b):
            return pltpu.make_async_copy(
                img_hbm.at[rows_a(i), :], spm.at[sid, b], ia_sems[b])

        def out_a(i, b):
            return pltpu.make_async_copy(
                spm.at[sid, b], out_hbm.at[rows_a(i), :], oa_sems[b])

        def in_b(i, b):
            return pltpu.make_async_copy(
                img_hbm.at[rows_b(i), :], tbufs[b], ib_sems[b])

        def out_b(i, b):
            return pltpu.make_async_copy(
                tbufs[b], out_hbm.at[rows_b(i), :], ob_sems[b])

        in_a(0, 0).start()
        in_b(0, 0).start()

        def body(h, carry):
            for b in range(2):
                g = h * 2 + b
                j = g + 1
                bj = 1 - b

                @pl.when(j < _NA)
                def _():
                    @pl.when(j >= 2)
                    def _():
                        out_a(j - 2, bj).wait()
                    in_a(j, bj).start()

                @pl.when(j < _NB)
                def _():
                    @pl.when(j >= 2)
                    def _():
                        out_b(j - 2, bj).wait()
                    in_b(j, bj).start()

                in_a(g, b).wait()
                for w, fbuf in zip(windows, fbufs):
                    pltpu.sync_copy(spm.at[sid, b, :, pl.ds(w, 128)], fbuf)

                def fix_a(r, carry2):
                    for w, fbuf in zip(windows, fbufs):
                        for gg in groups[w]:
                            v = fbuf[r, pl.ds(gg, 16)]
                            keep = jnp.ones((16,), jnp.float32)
                            for lane in lanes[w][gg]:
                                keep = jnp.where(iota == lane, 0.0, keep)
                            fbuf[r, pl.ds(gg, 16)] = v * keep
                    return carry2

                lax.fori_loop(0, _CH, fix_a, 0)
                for w, fbuf in zip(windows, fbufs):
                    pltpu.sync_copy(fbuf, spm.at[sid, b, :, pl.ds(w, 128)])
                out_a(g, b).start()

                in_b(g, b).wait()
                for r in range(_CH):
                    for c in disabled:
                        w16 = (c // 16) * 16
                        lane = c % 16
                        v = tbufs[b][r, pl.ds(w16, 16)]
                        tbufs[b][r, pl.ds(w16, 16)] = jnp.where(
                            iota == lane, 0.0, v)
                out_b(g, b).start()
            return carry

        lax.fori_loop(0, _NA // 2, body, 0)
        for b in range(2):
            out_a(_NA - 2 + b, b).wait()
            out_b(_NB - 2 + b, b).wait()

    return k


def kernel(img):
    return _build(img.shape[-1])(img)


# speedup vs baseline: 1.0580x; 1.0144x over previous
import functools

import jax
import jax.numpy as jnp
import numpy as np
from jax import lax
from jax.experimental import pallas as pl
from jax.experimental.pallas import tpu as pltpu
from jax.experimental.pallas import tpu_sc as plsc


def _disabled_tofs(tof_count, min_c, max_c, neighbor_p, seed=0):
    rng = np.random.default_rng(seed)
    count = int(rng.integers(min_c, max_c + 1))
    tof_list = rng.permutation(tof_count)
    first = int(rng.integers(1, tof_count))
    disabled = [first]
    tof_list = tof_list[tof_list != first]
    for _ in range(count - 1):
        r = float(rng.random())
        if r < neighbor_p:
            if r < neighbor_p / 2.0:
                offsets = (1, -1)
            else:
                offsets = (tof_count // 2, -(tof_count // 2))
            appended = False
            for d in list(disabled):
                for off in offsets:
                    cand = d + off
                    if cand in tof_list:
                        tof_list = tof_list[tof_list != cand]
                        disabled.append(int(cand))
                        appended = True
                        break
                if appended:
                    break
            if not appended:
                new = int(tof_list[0])
                tof_list = tof_list[tof_list != new]
                disabled.append(new)
        else:
            new = int(tof_list[0])
            tof_list = tof_list[tof_list != new]
            disabled.append(new)
    return sorted(int(x) for x in disabled)


_ROWS, _COLS = 16384, 2048
_NW = 32
_NS = 16
_CH = 16
_NBUF = 3
_RA = 1
_N = _ROWS // (_NW * _CH)
_PAD = _NBUF - (_N % _NBUF) if _N % _NBUF else 0


@functools.cache
def _build(tof_count):
    disabled = _disabled_tofs(tof_count, 1, 3, 0.5)
    windows = sorted({(c // 128) * 128 for c in disabled})
    groups = {w: sorted({((c - w) // 16) * 16 for c in disabled
                         if (c // 128) * 128 == w}) for w in windows}
    lanes = {w: {g: [c - w - g for c in disabled
                     if (c // 128) * 128 == w and ((c - w) // 16) * 16 == g]
                 for g in groups[w]} for w in windows}
    nwin = len(windows)
    mesh = plsc.VectorSubcoreMesh(core_axis_name="c", subcore_axis_name="s")

    @functools.partial(
        pl.kernel,
        mesh=mesh,
        out_type=jax.ShapeDtypeStruct((_ROWS, _COLS), jnp.float32),
        scratch_types=(
            [pltpu.VMEM_SHARED((_NS, _NBUF, _CH, _COLS), jnp.float32)]
            + [pltpu.VMEM((_CH, 128), jnp.float32) for _ in range(nwin)]
            + [pltpu.SemaphoreType.DMA for _ in range(2 * _NBUF)]
        ),
    )
    def k(img_hbm, out_hbm, spm, *rest):
        fbufs = rest[:nwin]
        isems = rest[nwin:nwin + _NBUF]
        osems = rest[nwin + _NBUF:nwin + 2 * _NBUF]
        sid = lax.axis_index("s")
        wid = sid * 2 + lax.axis_index("c")
        iota = lax.iota(jnp.int32, 16)

        def in_cp(i, b):
            r = pl.ds((i * _NW + wid) * _CH, _CH)
            return pltpu.make_async_copy(
                img_hbm.at[r, :], spm.at[sid, b], isems[b])

        def out_cp(i, b):
            r = pl.ds((i * _NW + wid) * _CH, _CH)
            return pltpu.make_async_copy(
                spm.at[sid, b], out_hbm.at[r, :], osems[b])

        for p in range(_RA):
            in_cp(p, p).start()

        def body(g, carry):
            for b in range(_NBUF):
                i = g * _NBUF + b
                j = i + _RA
                bj = (b + _RA) % _NBUF

                @pl.when(j < _N)
                def _():
                    @pl.when(j >= _NBUF)
                    def _():
                        out_cp(j - _NBUF, bj).wait()
                    in_cp(j, bj).start()

                @pl.when(i < _N)
                def _():
                    in_cp(i, b).wait()
                    for w, fbuf in zip(windows, fbufs):
                        pltpu.sync_copy(
                            spm.at[sid, b, :, pl.ds(w, 128)], fbuf)

                    def fix(r, carry2):
                        for w, fbuf in zip(windows, fbufs):
                            for gg in groups[w]:
                                v = fbuf[r, pl.ds(gg, 16)]
                                keep = jnp.ones((16,), jnp.float32)
                                for lane in lanes[w][gg]:
                                    keep = jnp.where(iota == lane, 0.0, keep)
                                fbuf[r, pl.ds(gg, 16)] = v * keep
                        return carry2

                    lax.fori_loop(0, _CH, fix, 0)
                    for w, fbuf in zip(windows, fbufs):
                        pltpu.sync_copy(
                            fbuf, spm.at[sid, b, :, pl.ds(w, 128)])
                    out_cp(i, b).start()
            return carry

        lax.fori_loop(0, (_N + _PAD) // _NBUF, body, 0)
        for b in range(_NBUF):
            last = _N - 1 - ((_N - 1 - b) % _NBUF)
            out_cp(last, last % _NBUF).wait()

    return k


def kernel(img):
    return _build(img.shape[-1])(img)
